# baseline (speedup 1.0000x reference)
import jax
import jax.numpy as jnp
from jax.experimental import pallas as pl

_TILE_L = 512


def _pe_kernel(table_ref, w_ref, b_ref, out_ref):
    proj = jnp.dot(table_ref[...], w_ref[...], preferred_element_type=jnp.float32)
    proj = proj + b_ref[...]
    out_ref[...] = proj[None, :, :]


def kernel(x, table, W, b):
    B, L, d_model = x.shape
    d_emb = table.shape[1]
    tile_l = min(_TILE_L, L)
    return pl.pallas_call(
        _pe_kernel,
        grid=(L // tile_l, B),
        in_specs=[
            pl.BlockSpec((tile_l, d_emb), lambda i, j: (i, 0)),
            pl.BlockSpec((d_emb, d_model), lambda i, j: (0, 0)),
            pl.BlockSpec((1, d_model), lambda i, j: (0, 0)),
        ],
        out_specs=pl.BlockSpec((1, tile_l, d_model), lambda i, j: (j, i, 0)),
        out_shape=jax.ShapeDtypeStruct((B, L, d_model), jnp.float32),
    )(table, W, b.reshape(1, d_model))


# speedup vs baseline: 1.3056x; 1.3056x over previous
import jax
import jax.numpy as jnp
from jax.experimental import pallas as pl

_TILE_L = 512


def _pe_kernel(table_ref, w_ref, b_ref, out_ref):
    proj = jnp.dot(table_ref[...], w_ref[...], preferred_element_type=jnp.float32)
    proj = proj + b_ref[...]
    out_ref[...] = jnp.broadcast_to(proj[None, :, :], out_ref.shape)


def kernel(x, table, W, b):
    B, L, d_model = x.shape
    d_emb = table.shape[1]
    tile_l = min(_TILE_L, L)
    return pl.pallas_call(
        _pe_kernel,
        grid=(L // tile_l,),
        in_specs=[
            pl.BlockSpec((tile_l, d_emb), lambda i: (i, 0)),
            pl.BlockSpec((d_emb, d_model), lambda i: (0, 0)),
            pl.BlockSpec((1, d_model), lambda i: (0, 0)),
        ],
        out_specs=pl.BlockSpec((B, tile_l, d_model), lambda i: (0, i, 0)),
        out_shape=jax.ShapeDtypeStruct((B, L, d_model), jnp.float32),
    )(table, W, b.reshape(1, d_model))
